# baseline (speedup 1.0000x reference)
import functools

import jax
import jax.numpy as jnp
from jax import lax
from jax.experimental import pallas as pl
from jax.experimental.pallas import tpu as pltpu
from jax.experimental.pallas import tpu_sc as plsc

_NC = 2
_NS = 16
_NW = _NC * _NS
_L = 16
_CHUNK = 128


def _rsqrt_newton(x):
    x = jnp.maximum(x, jnp.float32(1e-24))
    i = lax.bitcast_convert_type(x, jnp.int32)
    i = jnp.int32(0x5F3759DF) - lax.shift_right_arithmetic(i, 1)
    y = lax.bitcast_convert_type(i, jnp.float32)
    for _ in range(3):
        y = y * (jnp.float32(1.5) - jnp.float32(0.5) * x * y * y)
    return y


def kernel(h, r, t, entity_embed, relation_embed):
    B = h.shape[0]
    D = entity_embed.shape[1]
    bpw = B // _NW
    nch = bpw // _CHUNK
    ngrp = bpw // _L

    h3 = h.reshape(_NW, nch, _CHUNK)
    r3 = r.reshape(_NW, nch, _CHUNK)
    t3 = t.reshape(_NW, nch, _CHUNK)

    mesh = plsc.VectorSubcoreMesh(core_axis_name="c", subcore_axis_name="s")

    @functools.partial(
        pl.kernel,
        out_type=jax.ShapeDtypeStruct((B,), jnp.float32),
        mesh=mesh,
        compiler_params=pltpu.CompilerParams(
            needs_layout_passes=False, use_tc_tiling_on_sc=False),
        scratch_types=[
            pltpu.VMEM((nch, _CHUNK), jnp.int32),
            pltpu.VMEM((nch, _CHUNK), jnp.int32),
            pltpu.VMEM((nch, _CHUNK), jnp.int32),
            pltpu.VMEM((bpw, D), jnp.float32),
            pltpu.VMEM((bpw, D), jnp.float32),
            pltpu.VMEM((bpw, D), jnp.float32),
            pltpu.VMEM((bpw,), jnp.float32),
            pltpu.SemaphoreType.DMA,
        ],
    )
    def run(h_hbm, r_hbm, t_hbm, ent_hbm, rel_hbm, out_hbm,
            hi, ri, ti, hv, rv, tv, ov, sem):
        wid = lax.axis_index("s") * _NC + lax.axis_index("c")

        pltpu.sync_copy(h_hbm.at[wid], hi)
        pltpu.sync_copy(r_hbm.at[wid], ri)
        pltpu.sync_copy(t_hbm.at[wid], ti)

        copies = []
        for c in range(nch):
            dst = pl.ds(c * _CHUNK, _CHUNK)
            copies.append(pltpu.async_copy(ent_hbm.at[hi.at[c]], hv.at[dst], sem))
            copies.append(pltpu.async_copy(ent_hbm.at[ti.at[c]], tv.at[dst], sem))
            copies.append(pltpu.async_copy(rel_hbm.at[ri.at[c]], rv.at[dst], sem))
        for cp in copies:
            cp.wait()

        def group(g, carry):
            rows = g * _L + lax.iota(jnp.int32, _L)
            h2 = jnp.zeros((_L,), jnp.float32)
            t2 = jnp.zeros((_L,), jnp.float32)
            for j in range(D):
                cj = jnp.full((_L,), j, jnp.int32)
                hj = plsc.load_gather(hv, [rows, cj])
                tj = plsc.load_gather(tv, [rows, cj])
                h2 = h2 + hj * hj
                t2 = t2 + tj * tj
            ih = _rsqrt_newton(h2)
            it = _rsqrt_newton(t2)
            d = jnp.zeros((_L,), jnp.float32)
            for j in range(D):
                cj = jnp.full((_L,), j, jnp.int32)
                hj = plsc.load_gather(hv, [rows, cj])
                rj = plsc.load_gather(rv, [rows, cj])
                tj = plsc.load_gather(tv, [rows, cj])
                d = d + jnp.abs(hj * ih + rj - tj * it)
            ov[pl.ds(pl.multiple_of(g * _L, _L), _L)] = -d
            return carry

        lax.fori_loop(0, ngrp, group, 0)
        pltpu.sync_copy(ov, out_hbm.at[pl.ds(wid * bpw, bpw)])

    return run(h3, r3, t3, entity_embed, relation_embed)
